# baseline (speedup 1.0000x reference)
import functools

import jax
import jax.numpy as jnp
from jax import lax
from jax.experimental import pallas as pl
from jax.experimental.pallas import tpu as pltpu
from jax.experimental.pallas import tpu_sc as plsc

B, N1, N2, D = 4, 1024, 16384, 16
K = 8
L = 16
NW = 32
QPW = (B * N1) // NW
CN = 4096
NCHUNK = N2 // CN
NVEC = CN // L
CAP = 8

_INF = float("inf")
_IBIG = 2**30


def _bcast_i32(x):
    return jnp.full((L,), x, jnp.int32)


def _bcast_f32(x):
    return jnp.full((L,), x, jnp.float32)


def _lanes():
    return lax.broadcasted_iota(jnp.int32, (L,), 0)


def _bf16_round(v):
    bits = plsc.bitcast(v, jnp.int32)
    rounded = (bits + 0x7FFF + ((bits >> 16) & 1)) & ~0xFFFF
    return plsc.bitcast(rounded, jnp.float32)


def _rebuild(rows_v, rows_i):
    lanes = _lanes()
    tv = jnp.full((L,), _INF, jnp.float32)
    ti = jnp.full((L,), _IBIG, jnp.int32)
    t8s = jnp.float32(_INF)
    for r in range(K):
        mv = rows_v[0]
        for rv in rows_v[1:]:
            mv = jnp.minimum(mv, rv)
        m = jnp.min(mv)
        mb = _bcast_f32(m)
        sel = _bcast_i32(_IBIG)
        for rv, ri in zip(rows_v, rows_i):
            sel = jnp.minimum(sel, jnp.where(rv == mb, ri, _IBIG))
        j = jnp.min(sel)
        jb = _bcast_i32(j)
        rows_v = [jnp.where((rv == mb) & (ri == jb), _INF, rv)
                  for rv, ri in zip(rows_v, rows_i)]
        tv = jnp.where(lanes == r, mb, tv)
        ti = jnp.where(lanes == r, jb, ti)
        t8s = m
    return tv, ti, t8s


def _sc_body(p1_hbm, p2t_hbm, outd_hbm, outi_hbm,
             p1_v, p2t_v, p2sq_v, cand_v, cand_i, outd_v, outi_v):
    c = lax.axis_index("c")
    s = lax.axis_index("s")
    wid = s * 2 + c
    b = wid // (N1 // QPW)
    q0 = (wid % (N1 // QPW)) * QPW
    lanes = _lanes()

    pltpu.sync_copy(p1_hbm.at[pl.ds(wid * (QPW * D), QPW * D)], p1_v)

    def chunk_step(ch, _):
        k0 = ch * CN
        for d in range(D):
            pltpu.sync_copy(p2t_hbm.at[b * D + d, pl.ds(k0, CN)],
                            p2t_v.at[d])

        def sq_step(g, _):
            acc = jnp.zeros((L,), jnp.float32)
            for d in range(D):
                row = p2t_v[d, pl.ds(g * L, L)]
                acc = acc + row * row
                p2t_v[d, pl.ds(g * L, L)] = _bf16_round(row)
            p2sq_v[pl.ds(g * L, L)] = acc
            return 0

        lax.fori_loop(0, NVEC, sq_step, 0)

        def query_step(q, _):
            qrow = plsc.load_gather(
                p1_v, [_bcast_i32(q) * D + lanes])
            p1sqs = jnp.sum(qrow * qrow)
            bd = [_bf16_round(plsc.load_gather(p1_v, [_bcast_i32(q * D + d)]))
                  for d in range(D)]

            tv = outd_v[pl.ds(q * L, L)]
            ti = outi_v[pl.ds(q * L, L)]
            t8s = jnp.min(jnp.where(lanes == K - 1, tv, _INF))

            def flush(tv, ti):
                rows_v = [tv] + [cand_v[pl.ds(r * L, L)]
                                 for r in range(CAP)]
                rows_i = [ti] + [cand_i[pl.ds(r * L, L)]
                                 for r in range(CAP)]
                ntv, nti, nt8 = _rebuild(rows_v, rows_i)
                for r in range(CAP):
                    cand_v[pl.ds(r * L, L)] = _bcast_f32(_INF)
                    cand_i[pl.ds(r * L, L)] = _bcast_i32(_IBIG)
                return ntv, nti, nt8, jnp.int32(0)

            def vec_step(j, st):
                tv, ti, t8s, cnt = st
                acc = bd[0] * p2t_v[0, pl.ds(j * L, L)]
                for d in range(1, D):
                    acc = acc + bd[d] * p2t_v[d, pl.ds(j * L, L)]
                dv = (p1sqs + p2sq_v[pl.ds(j * L, L)]) - 2.0 * acc
                mn = jnp.min(dv)

                def hit(tv, ti, t8s, cnt):
                    cand_v[pl.ds(cnt * L, L)] = dv
                    cand_i[pl.ds(cnt * L, L)] = (k0 + j * L) + lanes
                    cnt1 = cnt + 1
                    return lax.cond(
                        cnt1 == CAP,
                        lambda tv, ti: flush(tv, ti),
                        lambda tv, ti: (tv, ti, t8s, cnt1),
                        tv, ti)

                return lax.cond(mn <= t8s, hit,
                                lambda tv, ti, t8s, cnt: (tv, ti, t8s, cnt),
                                tv, ti, t8s, cnt)

            tv, ti, t8s, cnt = lax.fori_loop(
                0, NVEC, vec_step, (tv, ti, t8s, jnp.int32(0)))

            tv, ti = lax.cond(
                cnt > 0,
                lambda tv, ti: flush(tv, ti)[:2],
                lambda tv, ti: (tv, ti), tv, ti)
            outd_v[pl.ds(q * L, L)] = tv
            outi_v[pl.ds(q * L, L)] = ti
            return 0

        lax.fori_loop(0, QPW, query_step, 0)
        return 0

    def init_step(q, _):
        outd_v[pl.ds(q * L, L)] = jnp.full((L,), _INF, jnp.float32)
        outi_v[pl.ds(q * L, L)] = jnp.full((L,), _IBIG, jnp.int32)
        return 0

    lax.fori_loop(0, QPW, init_step, 0)
    for r in range(CAP):
        cand_v[pl.ds(r * L, L)] = jnp.full((L,), _INF, jnp.float32)
        cand_i[pl.ds(r * L, L)] = jnp.full((L,), _IBIG, jnp.int32)

    lax.fori_loop(0, NCHUNK, chunk_step, 0)

    pltpu.sync_copy(outd_v, outd_hbm.at[pl.ds(wid * (QPW * L), QPW * L)])
    pltpu.sync_copy(outi_v, outi_hbm.at[pl.ds(wid * (QPW * L), QPW * L)])


def _sc_knn(p1, p2, interpret=False):
    p1f = p1.reshape(B * N1 * D)
    p2t = p2.transpose(0, 2, 1).reshape(B * D, N2)
    mesh = plsc.VectorSubcoreMesh(core_axis_name="c", subcore_axis_name="s",
                                  num_cores=2, num_subcores=16)
    f = pl.kernel(
        _sc_body,
        out_type=[
            jax.ShapeDtypeStruct((B * N1 * L,), jnp.float32),
            jax.ShapeDtypeStruct((B * N1 * L,), jnp.int32),
        ],
        mesh=mesh,
        scratch_types=[
            pltpu.VMEM((QPW * D,), jnp.float32),
            pltpu.VMEM((D, CN), jnp.float32),
            pltpu.VMEM((CN,), jnp.float32),
            pltpu.VMEM((CAP * L,), jnp.float32),
            pltpu.VMEM((CAP * L,), jnp.int32),
            pltpu.VMEM((QPW * L,), jnp.float32),
            pltpu.VMEM((QPW * L,), jnp.int32),
        ],
        compiler_params=pltpu.CompilerParams(needs_layout_passes=False),
        interpret=interpret,
    )
    outd, outi = f(p1f, p2t)
    outd = outd.reshape(B, N1, L)[:, :, :K]
    outi = outi.reshape(B, N1, L)[:, :, :K]
    return outd, outi


@jax.jit
def kernel(p1, p2):
    return _sc_knn(p1, p2)


# speedup vs baseline: 2.9199x; 2.9199x over previous
import jax
import jax.numpy as jnp
from jax import lax
from jax.experimental import pallas as pl
from jax.experimental.pallas import tpu as pltpu
from jax.experimental.pallas import tpu_sc as plsc

B, N1, N2, D = 4, 1024, 16384, 16
K = 8
L = 16
NW = 32
QPW = (B * N1) // NW
CN = 4096
NCHUNK = N2 // CN
NVEC = CN // L
G = 4
NGRP = NVEC // G
CAP = 8

_INF = float("inf")
_IBIG = 2**30


def _bcast_i32(x):
    return jnp.full((L,), x, jnp.int32)


def _bcast_f32(x):
    return jnp.full((L,), x, jnp.float32)


def _lanes():
    return lax.broadcasted_iota(jnp.int32, (L,), 0)


def _bf16_round(v):
    bits = plsc.bitcast(v, jnp.int32)
    rounded = (bits + 0x7FFF + ((bits >> 16) & 1)) & ~0xFFFF
    return plsc.bitcast(rounded, jnp.float32)


def _rebuild(rows_v, rows_i):
    lanes = _lanes()
    tv = jnp.full((L,), _INF, jnp.float32)
    ti = jnp.full((L,), _IBIG, jnp.int32)
    t8s = jnp.float32(_INF)
    for r in range(K):
        mv = rows_v[0]
        for rv in rows_v[1:]:
            mv = jnp.minimum(mv, rv)
        m = jnp.min(mv)
        mb = _bcast_f32(m)
        sel = _bcast_i32(_IBIG)
        for rv, ri in zip(rows_v, rows_i):
            sel = jnp.minimum(sel, jnp.where(rv == mb, ri, _IBIG))
        j = jnp.min(sel)
        jb = _bcast_i32(j)
        rows_v = [jnp.where((rv == mb) & (ri == jb), _INF, rv)
                  for rv, ri in zip(rows_v, rows_i)]
        tv = jnp.where(lanes == r, mb, tv)
        ti = jnp.where(lanes == r, jb, ti)
        t8s = m
    return tv, ti, t8s


def _sc_body(p1_hbm, p2t_hbm, outd_hbm, outi_hbm,
             p1_v, p2t_v, p2sq_v, cand_v, cand_i, outd_v, outi_v):
    c = lax.axis_index("c")
    s = lax.axis_index("s")
    wid = s * 2 + c
    b = wid // (N1 // QPW)
    lanes = _lanes()

    pltpu.sync_copy(p1_hbm.at[pl.ds(wid * (QPW * D), QPW * D)], p1_v)

    def init_step(q, _):
        outd_v[pl.ds(q * L, L)] = jnp.full((L,), _INF, jnp.float32)
        outi_v[pl.ds(q * L, L)] = jnp.full((L,), _IBIG, jnp.int32)
        return 0

    lax.fori_loop(0, QPW, init_step, 0)
    for r in range(2 * CAP):
        cand_v[pl.ds(r * L, L)] = jnp.full((L,), _INF, jnp.float32)
        cand_i[pl.ds(r * L, L)] = jnp.full((L,), _IBIG, jnp.int32)

    def chunk_step(ch, _):
        k0 = ch * CN
        for d in range(D):
            pltpu.sync_copy(p2t_hbm.at[b * D + d, pl.ds(k0, CN)],
                            p2t_v.at[d])

        def sq_step(g, _):
            acc = jnp.zeros((L,), jnp.float32)
            for d in range(D):
                row = p2t_v[d, pl.ds(g * L, L)]
                acc = acc + row * row
                p2t_v[d, pl.ds(g * L, L)] = _bf16_round(row)
            p2sq_v[pl.ds(g * L, L)] = acc
            return 0

        lax.fori_loop(0, NVEC, sq_step, 0)

        def pair_step(qi, _):
            qa = 2 * qi
            qb = 2 * qi + 1
            qrow_a = p1_v[pl.ds(qa * D, L)]
            qrow_b = p1_v[pl.ds(qb * D, L)]
            p1sq_a = jnp.sum(qrow_a * qrow_a)
            p1sq_b = jnp.sum(qrow_b * qrow_b)
            bda = [_bf16_round(plsc.load_gather(
                p1_v, [_bcast_i32(qa * D + d)])) for d in range(D)]
            bdb = [_bf16_round(plsc.load_gather(
                p1_v, [_bcast_i32(qb * D + d)])) for d in range(D)]

            tva0 = outd_v[pl.ds(qa * L, L)]
            tia0 = outi_v[pl.ds(qa * L, L)]
            tvb0 = outd_v[pl.ds(qb * L, L)]
            tib0 = outi_v[pl.ds(qb * L, L)]
            t8a0 = jnp.min(jnp.where(lanes == K - 1, tva0, _INF))
            t8b0 = jnp.min(jnp.where(lanes == K - 1, tvb0, _INF))

            def flush(sub, tv, ti):
                base = sub * CAP
                rows_v = [tv] + [cand_v[pl.ds((base + r) * L, L)]
                                 for r in range(CAP)]
                rows_i = [ti] + [cand_i[pl.ds((base + r) * L, L)]
                                 for r in range(CAP)]
                ntv, nti, nt8 = _rebuild(rows_v, rows_i)
                for r in range(CAP):
                    cand_v[pl.ds((base + r) * L, L)] = _bcast_f32(_INF)
                    cand_i[pl.ds((base + r) * L, L)] = _bcast_i32(_IBIG)
                return ntv, nti, nt8, jnp.int32(0)

            def grp_step(jg, st):
                tva, tia, t8a, cnta, tvb, tib, t8b, cntb = st
                dva = []
                dvb = []
                for r in range(G):
                    j = jg * G + r
                    acca = None
                    accb = None
                    for d in range(D):
                        row = p2t_v[d, pl.ds(j * L, L)]
                        if d == 0:
                            acca = bda[0] * row
                            accb = bdb[0] * row
                        else:
                            acca = acca + bda[d] * row
                            accb = accb + bdb[d] * row
                    p2sq = p2sq_v[pl.ds(j * L, L)]
                    dva.append((p1sq_a + p2sq) - 2.0 * acca)
                    dvb.append((p1sq_b + p2sq) - 2.0 * accb)
                mna = jnp.minimum(jnp.minimum(dva[0], dva[1]),
                                  jnp.minimum(dva[2], dva[3]))
                mnb = jnp.minimum(jnp.minimum(dvb[0], dvb[1]),
                                  jnp.minimum(dvb[2], dvb[3]))
                ma = jnp.min(mna)
                mb = jnp.min(mnb)

                def store_grp(sub, dv, cnt):
                    base = sub * CAP
                    for r in range(G):
                        off = (base + r) * L
                        cand_v[pl.ds(cnt * L + off, L)] = dv[r]
                        cand_i[pl.ds(cnt * L + off, L)] = (
                            (k0 + (jg * G + r) * L) + lanes)

                def hit(tva, tia, t8a, cnta, tvb, tib, t8b, cntb):
                    def hita(tv, ti):
                        store_grp(0, dva, cnta)
                        cnt1 = cnta + G
                        return lax.cond(
                            cnt1 == CAP,
                            lambda tv, ti: flush(0, tv, ti),
                            lambda tv, ti: (tv, ti, t8a, cnt1),
                            tv, ti)

                    def hitb(tv, ti):
                        store_grp(1, dvb, cntb)
                        cnt1 = cntb + G
                        return lax.cond(
                            cnt1 == CAP,
                            lambda tv, ti: flush(1, tv, ti),
                            lambda tv, ti: (tv, ti, t8b, cnt1),
                            tv, ti)

                    ra = lax.cond(ma <= t8a, hita,
                                  lambda tv, ti: (tv, ti, t8a, cnta),
                                  tva, tia)
                    rb = lax.cond(mb <= t8b, hitb,
                                  lambda tv, ti: (tv, ti, t8b, cntb),
                                  tvb, tib)
                    return ra + rb

                return lax.cond(
                    (ma <= t8a) | (mb <= t8b), hit,
                    lambda *st_: st_,
                    tva, tia, t8a, cnta, tvb, tib, t8b, cntb)

            st = lax.fori_loop(
                0, NGRP, grp_step,
                (tva0, tia0, t8a0, jnp.int32(0),
                 tvb0, tib0, t8b0, jnp.int32(0)))
            tva, tia, _, cnta, tvb, tib, _, cntb = st

            tva, tia = lax.cond(
                cnta > 0, lambda tv, ti: flush(0, tv, ti)[:2],
                lambda tv, ti: (tv, ti), tva, tia)
            tvb, tib = lax.cond(
                cntb > 0, lambda tv, ti: flush(1, tv, ti)[:2],
                lambda tv, ti: (tv, ti), tvb, tib)
            outd_v[pl.ds(qa * L, L)] = tva
            outi_v[pl.ds(qa * L, L)] = tia
            outd_v[pl.ds(qb * L, L)] = tvb
            outi_v[pl.ds(qb * L, L)] = tib
            return 0

        lax.fori_loop(0, QPW // 2, pair_step, 0)
        return 0

    lax.fori_loop(0, NCHUNK, chunk_step, 0)

    pltpu.sync_copy(outd_v, outd_hbm.at[pl.ds(wid * (QPW * L), QPW * L)])
    pltpu.sync_copy(outi_v, outi_hbm.at[pl.ds(wid * (QPW * L), QPW * L)])


def _sc_knn(p1, p2, interpret=False):
    p1f = p1.reshape(B * N1 * D)
    p2t = p2.transpose(0, 2, 1).reshape(B * D, N2)
    mesh = plsc.VectorSubcoreMesh(core_axis_name="c", subcore_axis_name="s",
                                  num_cores=2, num_subcores=16)
    f = pl.kernel(
        _sc_body,
        out_type=[
            jax.ShapeDtypeStruct((B * N1 * L,), jnp.float32),
            jax.ShapeDtypeStruct((B * N1 * L,), jnp.int32),
        ],
        mesh=mesh,
        scratch_types=[
            pltpu.VMEM((QPW * D,), jnp.float32),
            pltpu.VMEM((D, CN), jnp.float32),
            pltpu.VMEM((CN,), jnp.float32),
            pltpu.VMEM((2 * CAP * L,), jnp.float32),
            pltpu.VMEM((2 * CAP * L,), jnp.int32),
            pltpu.VMEM((QPW * L,), jnp.float32),
            pltpu.VMEM((QPW * L,), jnp.int32),
        ],
        compiler_params=pltpu.CompilerParams(needs_layout_passes=False),
        interpret=interpret,
    )
    outd, outi = f(p1f, p2t)
    outd = outd.reshape(B, N1, L)[:, :, :K]
    outi = outi.reshape(B, N1, L)[:, :, :K]
    return outd, outi


@jax.jit
def kernel(p1, p2):
    return _sc_knn(p1, p2)


# speedup vs baseline: 9.1990x; 3.1505x over previous
import jax
import jax.numpy as jnp
from jax import lax
from jax.experimental import pallas as pl
from jax.experimental.pallas import tpu as pltpu
from jax.experimental.pallas import tpu_sc as plsc

B, N1, N2, D = 4, 1024, 16384, 16
K = 8
SPLIT = 768

BQ = 256
CK = 2048

_INF = float("inf")
_IBIG = 2**30


def _select_topk(vals, idx, n_rounds):
    out_v = []
    out_i = []
    for _ in range(n_rounds):
        m = jnp.min(vals, axis=1, keepdims=True)
        sel = jnp.where(vals == m, idx, _IBIG)
        j = jnp.min(sel, axis=1, keepdims=True)
        vals = jnp.where(sel == j, _INF, vals)
        out_v.append(m)
        out_i.append(j)
    return jnp.concatenate(out_v, axis=1), jnp.concatenate(out_i, axis=1)


def _knn_body(p1_ref, p2_ref, dist_ref, idx_ref):
    p1b = p1_ref[0]
    p1_sq = jnp.sum(p1b * p1b, axis=1, keepdims=True)
    lane = lax.broadcasted_iota(jnp.int32, (BQ, CK), 1)
    n2 = p2_ref.shape[1]

    def chunk_step(c, carry):
        run_v, run_i = carry
        p2c = p2_ref[0, pl.ds(c * CK, CK), :]
        inner = lax.dot_general(
            p1b, p2c,
            dimension_numbers=(((1,), (1,)), ((), ())),
            preferred_element_type=jnp.float32,
        )
        p2_sq = jnp.sum(p2c * p2c, axis=1)[None, :]
        d = p1_sq + p2_sq - 2.0 * inner
        gidx = c * CK + lane
        cv, ci = _select_topk(d, gidx, K)
        ev = jnp.concatenate([run_v, cv], axis=1)
        ei = jnp.concatenate([run_i, ci], axis=1)
        return _select_topk(ev, ei, K)

    init = (jnp.full((BQ, K), _INF, jnp.float32),
            jnp.full((BQ, K), _IBIG, jnp.int32))
    run_v, run_i = lax.fori_loop(0, n2 // CK, chunk_step, init)
    dist_ref[0] = run_v
    idx_ref[0] = run_i


def _tc_knn(p1, p2):
    b, n1, d = p1.shape
    _, n2, _ = p2.shape
    grid = (b, n1 // BQ)
    return pl.pallas_call(
        _knn_body,
        grid=grid,
        in_specs=[
            pl.BlockSpec((1, BQ, d), lambda i, j: (i, j, 0)),
            pl.BlockSpec((1, n2, d), lambda i, j: (i, 0, 0)),
        ],
        out_specs=[
            pl.BlockSpec((1, BQ, K), lambda i, j: (i, j, 0)),
            pl.BlockSpec((1, BQ, K), lambda i, j: (i, j, 0)),
        ],
        out_shape=[
            jax.ShapeDtypeStruct((b, n1, K), jnp.float32),
            jax.ShapeDtypeStruct((b, n1, K), jnp.int32),
        ],
    )(p1, p2)


L = 16
NW = 32
QPW = (B * (N1 - SPLIT)) // NW
CN = 4096
NCHUNK = N2 // CN
NVEC = CN // L
G = 4
NGRP = NVEC // G
CAP = 8


def _bcast_i32(x):
    return jnp.full((L,), x, jnp.int32)


def _bcast_f32(x):
    return jnp.full((L,), x, jnp.float32)


def _lanes():
    return lax.broadcasted_iota(jnp.int32, (L,), 0)


def _bf16_round(v):
    bits = plsc.bitcast(v, jnp.int32)
    rounded = (bits + 0x7FFF + ((bits >> 16) & 1)) & ~0xFFFF
    return plsc.bitcast(rounded, jnp.float32)


def _rebuild(rows_v, rows_i):
    lanes = _lanes()
    tv = jnp.full((L,), _INF, jnp.float32)
    ti = jnp.full((L,), _IBIG, jnp.int32)
    t8s = jnp.float32(_INF)
    for r in range(K):
        mv = rows_v[0]
        for rv in rows_v[1:]:
            mv = jnp.minimum(mv, rv)
        m = jnp.min(mv)
        mb = _bcast_f32(m)
        sel = _bcast_i32(_IBIG)
        for rv, ri in zip(rows_v, rows_i):
            sel = jnp.minimum(sel, jnp.where(rv == mb, ri, _IBIG))
        j = jnp.min(sel)
        jb = _bcast_i32(j)
        rows_v = [jnp.where((rv == mb) & (ri == jb), _INF, rv)
                  for rv, ri in zip(rows_v, rows_i)]
        tv = jnp.where(lanes == r, mb, tv)
        ti = jnp.where(lanes == r, jb, ti)
        t8s = m
    return tv, ti, t8s


def _sc_body(p1_hbm, p2t_hbm, outd_hbm, outi_hbm,
             p1_v, p2t_v, p2sq_v, cand_v, cand_i, outd_v, outi_v):
    c = lax.axis_index("c")
    s = lax.axis_index("s")
    wid = s * 2 + c
    b = wid // ((N1 - SPLIT) // QPW)
    lanes = _lanes()

    pltpu.sync_copy(p1_hbm.at[pl.ds(wid * (QPW * D), QPW * D)], p1_v)

    def init_step(q, _):
        outd_v[pl.ds(q * L, L)] = jnp.full((L,), _INF, jnp.float32)
        outi_v[pl.ds(q * L, L)] = jnp.full((L,), _IBIG, jnp.int32)
        return 0

    lax.fori_loop(0, QPW, init_step, 0)
    for r in range(2 * CAP):
        cand_v[pl.ds(r * L, L)] = jnp.full((L,), _INF, jnp.float32)
        cand_i[pl.ds(r * L, L)] = jnp.full((L,), _IBIG, jnp.int32)

    def chunk_step(ch, _):
        k0 = ch * CN
        for d in range(D):
            pltpu.sync_copy(p2t_hbm.at[b * D + d, pl.ds(k0, CN)],
                            p2t_v.at[d])

        def sq_step(g, _):
            acc = jnp.zeros((L,), jnp.float32)
            for d in range(D):
                row = p2t_v[d, pl.ds(g * L, L)]
                acc = acc + row * row
                p2t_v[d, pl.ds(g * L, L)] = _bf16_round(row)
            p2sq_v[pl.ds(g * L, L)] = acc
            return 0

        lax.fori_loop(0, NVEC, sq_step, 0)

        def pair_step(qi, _):
            qa = 2 * qi
            qb = 2 * qi + 1
            qrow_a = p1_v[pl.ds(qa * D, L)]
            qrow_b = p1_v[pl.ds(qb * D, L)]
            p1sq_a = jnp.sum(qrow_a * qrow_a)
            p1sq_b = jnp.sum(qrow_b * qrow_b)
            bda = [_bf16_round(plsc.load_gather(
                p1_v, [_bcast_i32(qa * D + d)])) for d in range(D)]
            bdb = [_bf16_round(plsc.load_gather(
                p1_v, [_bcast_i32(qb * D + d)])) for d in range(D)]

            tva0 = outd_v[pl.ds(qa * L, L)]
            tia0 = outi_v[pl.ds(qa * L, L)]
            tvb0 = outd_v[pl.ds(qb * L, L)]
            tib0 = outi_v[pl.ds(qb * L, L)]
            t8a0 = jnp.min(jnp.where(lanes == K - 1, tva0, _INF))
            t8b0 = jnp.min(jnp.where(lanes == K - 1, tvb0, _INF))

            def flush(sub, tv, ti):
                base = sub * CAP
                rows_v = [tv] + [cand_v[pl.ds((base + r) * L, L)]
                                 for r in range(CAP)]
                rows_i = [ti] + [cand_i[pl.ds((base + r) * L, L)]
                                 for r in range(CAP)]
                ntv, nti, nt8 = _rebuild(rows_v, rows_i)
                for r in range(CAP):
                    cand_v[pl.ds((base + r) * L, L)] = _bcast_f32(_INF)
                    cand_i[pl.ds((base + r) * L, L)] = _bcast_i32(_IBIG)
                return ntv, nti, nt8, jnp.int32(0)

            def grp_step(jg, st):
                tva, tia, t8a, cnta, tvb, tib, t8b, cntb = st
                dva = []
                dvb = []
                for r in range(G):
                    j = jg * G + r
                    acca = None
                    accb = None
                    for d in range(D):
                        row = p2t_v[d, pl.ds(j * L, L)]
                        if d == 0:
                            acca = bda[0] * row
                            accb = bdb[0] * row
                        else:
                            acca = acca + bda[d] * row
                            accb = accb + bdb[d] * row
                    p2sq = p2sq_v[pl.ds(j * L, L)]
                    dva.append((p1sq_a + p2sq) - 2.0 * acca)
                    dvb.append((p1sq_b + p2sq) - 2.0 * accb)
                mna = jnp.minimum(jnp.minimum(dva[0], dva[1]),
                                  jnp.minimum(dva[2], dva[3]))
                mnb = jnp.minimum(jnp.minimum(dvb[0], dvb[1]),
                                  jnp.minimum(dvb[2], dvb[3]))
                ma = jnp.min(mna)
                mb = jnp.min(mnb)

                def store_grp(sub, dv, cnt):
                    base = sub * CAP
                    for r in range(G):
                        off = (base + r) * L
                        cand_v[pl.ds(cnt * L + off, L)] = dv[r]
                        cand_i[pl.ds(cnt * L + off, L)] = (
                            (k0 + (jg * G + r) * L) + lanes)

                def hit(tva, tia, t8a, cnta, tvb, tib, t8b, cntb):
                    def hita(tv, ti):
                        store_grp(0, dva, cnta)
                        cnt1 = cnta + G
                        return lax.cond(
                            cnt1 == CAP,
                            lambda tv, ti: flush(0, tv, ti),
                            lambda tv, ti: (tv, ti, t8a, cnt1),
                            tv, ti)

                    def hitb(tv, ti):
                        store_grp(1, dvb, cntb)
                        cnt1 = cntb + G
                        return lax.cond(
                            cnt1 == CAP,
                            lambda tv, ti: flush(1, tv, ti),
                            lambda tv, ti: (tv, ti, t8b, cnt1),
                            tv, ti)

                    ra = lax.cond(ma <= t8a, hita,
                                  lambda tv, ti: (tv, ti, t8a, cnta),
                                  tva, tia)
                    rb = lax.cond(mb <= t8b, hitb,
                                  lambda tv, ti: (tv, ti, t8b, cntb),
                                  tvb, tib)
                    return ra + rb

                return lax.cond(
                    (ma <= t8a) | (mb <= t8b), hit,
                    lambda *st_: st_,
                    tva, tia, t8a, cnta, tvb, tib, t8b, cntb)

            st = lax.fori_loop(
                0, NGRP, grp_step,
                (tva0, tia0, t8a0, jnp.int32(0),
                 tvb0, tib0, t8b0, jnp.int32(0)))
            tva, tia, _, cnta, tvb, tib, _, cntb = st

            tva, tia = lax.cond(
                cnta > 0, lambda tv, ti: flush(0, tv, ti)[:2],
                lambda tv, ti: (tv, ti), tva, tia)
            tvb, tib = lax.cond(
                cntb > 0, lambda tv, ti: flush(1, tv, ti)[:2],
                lambda tv, ti: (tv, ti), tvb, tib)
            outd_v[pl.ds(qa * L, L)] = tva
            outi_v[pl.ds(qa * L, L)] = tia
            outd_v[pl.ds(qb * L, L)] = tvb
            outi_v[pl.ds(qb * L, L)] = tib
            return 0

        lax.fori_loop(0, QPW // 2, pair_step, 0)
        return 0

    lax.fori_loop(0, NCHUNK, chunk_step, 0)

    pltpu.sync_copy(outd_v, outd_hbm.at[pl.ds(wid * (QPW * L), QPW * L)])
    pltpu.sync_copy(outi_v, outi_hbm.at[pl.ds(wid * (QPW * L), QPW * L)])


def _sc_knn(p1_tail, p2):
    p1f = p1_tail.reshape(B * (N1 - SPLIT) * D)
    p2t = p2.transpose(0, 2, 1).reshape(B * D, N2)
    mesh = plsc.VectorSubcoreMesh(core_axis_name="c", subcore_axis_name="s",
                                  num_cores=2, num_subcores=16)
    f = pl.kernel(
        _sc_body,
        out_type=[
            jax.ShapeDtypeStruct((B * (N1 - SPLIT) * L,), jnp.float32),
            jax.ShapeDtypeStruct((B * (N1 - SPLIT) * L,), jnp.int32),
        ],
        mesh=mesh,
        scratch_types=[
            pltpu.VMEM((QPW * D,), jnp.float32),
            pltpu.VMEM((D, CN), jnp.float32),
            pltpu.VMEM((CN,), jnp.float32),
            pltpu.VMEM((2 * CAP * L,), jnp.float32),
            pltpu.VMEM((2 * CAP * L,), jnp.int32),
            pltpu.VMEM((QPW * L,), jnp.float32),
            pltpu.VMEM((QPW * L,), jnp.int32),
        ],
        compiler_params=pltpu.CompilerParams(needs_layout_passes=False),
    )
    outd, outi = f(p1f, p2t)
    outd = outd.reshape(B, N1 - SPLIT, L)[:, :, :K]
    outi = outi.reshape(B, N1 - SPLIT, L)[:, :, :K]
    return outd, outi


@jax.jit
def kernel(p1, p2):
    sc_d, sc_i = _sc_knn(p1[:, SPLIT:, :], p2)
    tc_d, tc_i = _tc_knn(p1[:, :SPLIT, :], p2)
    dists = jnp.concatenate([tc_d, sc_d], axis=1)
    idx = jnp.concatenate([tc_i, sc_i], axis=1)
    return dists, idx


# speedup vs baseline: 10.0981x; 1.0977x over previous
import jax
import jax.numpy as jnp
from jax import lax
from jax.experimental import pallas as pl
from jax.experimental.pallas import tpu as pltpu
from jax.experimental.pallas import tpu_sc as plsc

B, N1, N2, D = 4, 1024, 16384, 16
K = 8
SPLIT = 768

BQ = 256
CK = 4096

_INF = float("inf")
_IBIG = 2**30


def _select_topk(vals, idx, n_rounds):
    out_v = []
    out_i = []
    for _ in range(n_rounds):
        m = jnp.min(vals, axis=1, keepdims=True)
        sel = jnp.where(vals == m, idx, _IBIG)
        j = jnp.min(sel, axis=1, keepdims=True)
        vals = jnp.where(sel == j, _INF, vals)
        out_v.append(m)
        out_i.append(j)
    return jnp.concatenate(out_v, axis=1), jnp.concatenate(out_i, axis=1)


def _knn_body(p1_ref, p2_ref, dist_ref, idx_ref):
    p1b = p1_ref[0]
    p1_sq = jnp.sum(p1b * p1b, axis=1, keepdims=True)
    lane = lax.broadcasted_iota(jnp.int32, (BQ, CK), 1)
    n2 = p2_ref.shape[1]

    def chunk_step(c, carry):
        run_v, run_i = carry
        p2c = p2_ref[0, pl.ds(c * CK, CK), :]
        inner = lax.dot_general(
            p1b, p2c,
            dimension_numbers=(((1,), (1,)), ((), ())),
            preferred_element_type=jnp.float32,
        )
        p2_sq = jnp.sum(p2c * p2c, axis=1)[None, :]
        d = p1_sq + p2_sq - 2.0 * inner
        gidx = c * CK + lane
        cv, ci = _select_topk(d, gidx, K)
        ev = jnp.concatenate([run_v, cv], axis=1)
        ei = jnp.concatenate([run_i, ci], axis=1)
        return _select_topk(ev, ei, K)

    init = (jnp.full((BQ, K), _INF, jnp.float32),
            jnp.full((BQ, K), _IBIG, jnp.int32))
    run_v, run_i = lax.fori_loop(0, n2 // CK, chunk_step, init)
    dist_ref[0] = run_v
    idx_ref[0] = run_i


def _tc_knn(p1, p2):
    b, n1, d = p1.shape
    _, n2, _ = p2.shape
    grid = (b, n1 // BQ)
    return pl.pallas_call(
        _knn_body,
        grid=grid,
        in_specs=[
            pl.BlockSpec((1, BQ, d), lambda i, j: (i, j, 0)),
            pl.BlockSpec((1, n2, d), lambda i, j: (i, 0, 0)),
        ],
        out_specs=[
            pl.BlockSpec((1, BQ, K), lambda i, j: (i, j, 0)),
            pl.BlockSpec((1, BQ, K), lambda i, j: (i, j, 0)),
        ],
        out_shape=[
            jax.ShapeDtypeStruct((b, n1, K), jnp.float32),
            jax.ShapeDtypeStruct((b, n1, K), jnp.int32),
        ],
    )(p1, p2)


L = 16
NW = 32
QPW = (B * (N1 - SPLIT)) // NW
CN = 4096
NCHUNK = N2 // CN
NVEC = CN // L
G = 4
NGRP = NVEC // G
CAP = 8


def _bcast_i32(x):
    return jnp.full((L,), x, jnp.int32)


def _bcast_f32(x):
    return jnp.full((L,), x, jnp.float32)


def _lanes():
    return lax.broadcasted_iota(jnp.int32, (L,), 0)


def _bf16_round(v):
    bits = plsc.bitcast(v, jnp.int32)
    rounded = (bits + 0x7FFF + ((bits >> 16) & 1)) & ~0xFFFF
    return plsc.bitcast(rounded, jnp.float32)


def _rebuild(rows_v, rows_i):
    lanes = _lanes()
    tv = jnp.full((L,), _INF, jnp.float32)
    ti = jnp.full((L,), _IBIG, jnp.int32)
    t8s = jnp.float32(_INF)
    for r in range(K):
        mv = rows_v[0]
        for rv in rows_v[1:]:
            mv = jnp.minimum(mv, rv)
        m = jnp.min(mv)
        mb = _bcast_f32(m)
        sel = _bcast_i32(_IBIG)
        for rv, ri in zip(rows_v, rows_i):
            sel = jnp.minimum(sel, jnp.where(rv == mb, ri, _IBIG))
        j = jnp.min(sel)
        jb = _bcast_i32(j)
        rows_v = [jnp.where((rv == mb) & (ri == jb), _INF, rv)
                  for rv, ri in zip(rows_v, rows_i)]
        tv = jnp.where(lanes == r, mb, tv)
        ti = jnp.where(lanes == r, jb, ti)
        t8s = m
    return tv, ti, t8s


def _sc_body(p1_hbm, p2t_hbm, outd_hbm, outi_hbm,
             p1_v, p2t_v, p2sq_v, cand_v, cand_i, outd_v, outi_v):
    c = lax.axis_index("c")
    s = lax.axis_index("s")
    wid = s * 2 + c
    b = wid // ((N1 - SPLIT) // QPW)
    lanes = _lanes()

    pltpu.sync_copy(p1_hbm.at[pl.ds(wid * (QPW * D), QPW * D)], p1_v)

    def init_step(q, _):
        outd_v[pl.ds(q * L, L)] = jnp.full((L,), _INF, jnp.float32)
        outi_v[pl.ds(q * L, L)] = jnp.full((L,), _IBIG, jnp.int32)
        return 0

    lax.fori_loop(0, QPW, init_step, 0)
    for r in range(2 * CAP):
        cand_v[pl.ds(r * L, L)] = jnp.full((L,), _INF, jnp.float32)
        cand_i[pl.ds(r * L, L)] = jnp.full((L,), _IBIG, jnp.int32)

    def chunk_step(ch, _):
        k0 = ch * CN
        for d in range(D):
            pltpu.sync_copy(p2t_hbm.at[b * D + d, pl.ds(k0, CN)],
                            p2t_v.at[d])

        def sq_step(g, _):
            acc = jnp.zeros((L,), jnp.float32)
            for d in range(D):
                row = p2t_v[d, pl.ds(g * L, L)]
                acc = acc + row * row
                p2t_v[d, pl.ds(g * L, L)] = _bf16_round(row)
            p2sq_v[pl.ds(g * L, L)] = acc
            return 0

        lax.fori_loop(0, NVEC, sq_step, 0)

        def pair_step(qi, _):
            qa = 2 * qi
            qb = 2 * qi + 1
            qrow_a = p1_v[pl.ds(qa * D, L)]
            qrow_b = p1_v[pl.ds(qb * D, L)]
            p1sq_a = jnp.sum(qrow_a * qrow_a)
            p1sq_b = jnp.sum(qrow_b * qrow_b)
            bda = [_bf16_round(plsc.load_gather(
                p1_v, [_bcast_i32(qa * D + d)])) for d in range(D)]
            bdb = [_bf16_round(plsc.load_gather(
                p1_v, [_bcast_i32(qb * D + d)])) for d in range(D)]

            tva0 = outd_v[pl.ds(qa * L, L)]
            tia0 = outi_v[pl.ds(qa * L, L)]
            tvb0 = outd_v[pl.ds(qb * L, L)]
            tib0 = outi_v[pl.ds(qb * L, L)]
            t8a0 = jnp.min(jnp.where(lanes == K - 1, tva0, _INF))
            t8b0 = jnp.min(jnp.where(lanes == K - 1, tvb0, _INF))

            def flush(sub, tv, ti):
                base = sub * CAP
                rows_v = [tv] + [cand_v[pl.ds((base + r) * L, L)]
                                 for r in range(CAP)]
                rows_i = [ti] + [cand_i[pl.ds((base + r) * L, L)]
                                 for r in range(CAP)]
                ntv, nti, nt8 = _rebuild(rows_v, rows_i)
                for r in range(CAP):
                    cand_v[pl.ds((base + r) * L, L)] = _bcast_f32(_INF)
                    cand_i[pl.ds((base + r) * L, L)] = _bcast_i32(_IBIG)
                return ntv, nti, nt8, jnp.int32(0)

            def grp_step(jg, st):
                tva, tia, t8a, cnta, tvb, tib, t8b, cntb = st
                dva = []
                dvb = []
                for r in range(G):
                    j = jg * G + r
                    acca = None
                    accb = None
                    for d in range(D):
                        row = p2t_v[d, pl.ds(j * L, L)]
                        if d == 0:
                            acca = bda[0] * row
                            accb = bdb[0] * row
                        else:
                            acca = acca + bda[d] * row
                            accb = accb + bdb[d] * row
                    p2sq = p2sq_v[pl.ds(j * L, L)]
                    dva.append((p1sq_a + p2sq) - 2.0 * acca)
                    dvb.append((p1sq_b + p2sq) - 2.0 * accb)
                mna = jnp.minimum(jnp.minimum(dva[0], dva[1]),
                                  jnp.minimum(dva[2], dva[3]))
                mnb = jnp.minimum(jnp.minimum(dvb[0], dvb[1]),
                                  jnp.minimum(dvb[2], dvb[3]))
                ma = jnp.min(mna)
                mb = jnp.min(mnb)

                def store_grp(sub, dv, cnt):
                    base = sub * CAP
                    for r in range(G):
                        off = (base + r) * L
                        cand_v[pl.ds(cnt * L + off, L)] = dv[r]
                        cand_i[pl.ds(cnt * L + off, L)] = (
                            (k0 + (jg * G + r) * L) + lanes)

                def hit(tva, tia, t8a, cnta, tvb, tib, t8b, cntb):
                    def hita(tv, ti):
                        store_grp(0, dva, cnta)
                        cnt1 = cnta + G
                        return lax.cond(
                            cnt1 == CAP,
                            lambda tv, ti: flush(0, tv, ti),
                            lambda tv, ti: (tv, ti, t8a, cnt1),
                            tv, ti)

                    def hitb(tv, ti):
                        store_grp(1, dvb, cntb)
                        cnt1 = cntb + G
                        return lax.cond(
                            cnt1 == CAP,
                            lambda tv, ti: flush(1, tv, ti),
                            lambda tv, ti: (tv, ti, t8b, cnt1),
                            tv, ti)

                    ra = lax.cond(ma <= t8a, hita,
                                  lambda tv, ti: (tv, ti, t8a, cnta),
                                  tva, tia)
                    rb = lax.cond(mb <= t8b, hitb,
                                  lambda tv, ti: (tv, ti, t8b, cntb),
                                  tvb, tib)
                    return ra + rb

                return lax.cond(
                    (ma <= t8a) | (mb <= t8b), hit,
                    lambda *st_: st_,
                    tva, tia, t8a, cnta, tvb, tib, t8b, cntb)

            st = lax.fori_loop(
                0, NGRP, grp_step,
                (tva0, tia0, t8a0, jnp.int32(0),
                 tvb0, tib0, t8b0, jnp.int32(0)))
            tva, tia, _, cnta, tvb, tib, _, cntb = st

            tva, tia = lax.cond(
                cnta > 0, lambda tv, ti: flush(0, tv, ti)[:2],
                lambda tv, ti: (tv, ti), tva, tia)
            tvb, tib = lax.cond(
                cntb > 0, lambda tv, ti: flush(1, tv, ti)[:2],
                lambda tv, ti: (tv, ti), tvb, tib)
            outd_v[pl.ds(qa * L, L)] = tva
            outi_v[pl.ds(qa * L, L)] = tia
            outd_v[pl.ds(qb * L, L)] = tvb
            outi_v[pl.ds(qb * L, L)] = tib
            return 0

        lax.fori_loop(0, QPW // 2, pair_step, 0)
        return 0

    lax.fori_loop(0, NCHUNK, chunk_step, 0)

    pltpu.sync_copy(outd_v, outd_hbm.at[pl.ds(wid * (QPW * L), QPW * L)])
    pltpu.sync_copy(outi_v, outi_hbm.at[pl.ds(wid * (QPW * L), QPW * L)])


def _sc_knn(p1_tail, p2):
    p1f = p1_tail.reshape(B * (N1 - SPLIT) * D)
    p2t = p2.transpose(0, 2, 1).reshape(B * D, N2)
    mesh = plsc.VectorSubcoreMesh(core_axis_name="c", subcore_axis_name="s",
                                  num_cores=2, num_subcores=16)
    f = pl.kernel(
        _sc_body,
        out_type=[
            jax.ShapeDtypeStruct((B * (N1 - SPLIT) * L,), jnp.float32),
            jax.ShapeDtypeStruct((B * (N1 - SPLIT) * L,), jnp.int32),
        ],
        mesh=mesh,
        scratch_types=[
            pltpu.VMEM((QPW * D,), jnp.float32),
            pltpu.VMEM((D, CN), jnp.float32),
            pltpu.VMEM((CN,), jnp.float32),
            pltpu.VMEM((2 * CAP * L,), jnp.float32),
            pltpu.VMEM((2 * CAP * L,), jnp.int32),
            pltpu.VMEM((QPW * L,), jnp.float32),
            pltpu.VMEM((QPW * L,), jnp.int32),
        ],
        compiler_params=pltpu.CompilerParams(needs_layout_passes=False),
    )
    outd, outi = f(p1f, p2t)
    outd = outd.reshape(B, N1 - SPLIT, L)[:, :, :K]
    outi = outi.reshape(B, N1 - SPLIT, L)[:, :, :K]
    return outd, outi


@jax.jit
def kernel(p1, p2):
    sc_d, sc_i = _sc_knn(p1[:, SPLIT:, :], p2)
    tc_d, tc_i = _tc_knn(p1[:, :SPLIT, :], p2)
    dists = jnp.concatenate([tc_d, sc_d], axis=1)
    idx = jnp.concatenate([tc_i, sc_i], axis=1)
    return dists, idx
